# baseline (speedup 1.0000x reference)
import functools

import jax
import jax.numpy as jnp
from jax import lax
from jax.experimental import pallas as pl
from jax.experimental.pallas import tpu as pltpu
from jax.experimental.pallas import tpu_sc as plsc

BATCH = 4
SEQ = 2048
HIDDEN = 1024
NUM_TYPES = 2
LANES = 16
NJ = HIDDEN // LANES

ROWS = BATCH * SEQ
NW = 32
ROWS_PER_W = ROWS // NW
S_PER_W = SEQ // NW
CHUNK = 8
NCHUNKS = ROWS_PER_W // CHUNK
SCH_PER_B = S_PER_W // CHUNK
NBUF = 4


def _emb_body(ids_hbm, ttids_hbm, table_hbm, tt2_hbm, pos_hbm, out_hbm,
              idx_all, tid_all, pos_v, tok0, tok1, tok2, tok3, tt2_v,
              g0, g1, g2, g3, o0, o1, o2, o3):
    nc = lax.axis_size("c")
    wid = lax.axis_index("s") * nc + lax.axis_index("c")
    sbase = wid * S_PER_W

    toks = [tok0, tok1, tok2, tok3]
    gsems = [g0, g1, g2, g3]
    osems = [o0, o1, o2, o3]

    pltpu.sync_copy(pos_hbm.at[pl.ds(sbase, S_PER_W)], pos_v)
    for b in range(BATCH):
        off = b * SEQ + sbase
        pltpu.sync_copy(ids_hbm.at[pl.ds(off, S_PER_W)],
                        idx_all.at[pl.ds(b * S_PER_W, S_PER_W)])
        pltpu.sync_copy(ttids_hbm.at[pl.ds(off, S_PER_W)],
                        tid_all.at[pl.ds(b * S_PER_W, S_PER_W)])
    pltpu.sync_copy(tt2_hbm, tt2_v)
    for j in range(NJ):
        dsl = pl.ds(j * LANES, LANES)
        tt2_v[1, dsl] = tt2_v[1, dsl] - tt2_v[0, dsl]

    def gather_start(cur, q):
        pltpu.make_async_copy(
            table_hbm.at[idx_all.at[pl.ds(cur * CHUNK, CHUNK)]],
            toks[q], gsems[q]).start()

    def out_wait(q):
        pltpu.make_async_copy(toks[q], out_hbm.at[pl.ds(0, CHUNK)],
                              osems[q]).wait()

    def step(cur, k):
        @pl.when(cur + (NBUF - 1) < NCHUNKS)
        def _():
            @pl.when(cur >= 1)
            def _():
                out_wait((k + NBUF - 1) % NBUF)
            gather_start(cur + (NBUF - 1), (k + NBUF - 1) % NBUF)

        pltpu.make_async_copy(
            table_hbm.at[idx_all.at[pl.ds(0, CHUNK)]],
            toks[k], gsems[k]).wait()

        b = cur // SCH_PER_B
        t = lax.rem(cur, SCH_PER_B)
        tokq = toks[k]
        ttf = tid_all[pl.ds((cur // 2) * LANES, LANES)].astype(jnp.float32)
        lbase = lax.rem(cur, 2) * CHUNK
        fvecs = [
            ttf.at[jnp.full((LANES,), lbase + r, jnp.int32)].get(
                mode="promise_in_bounds") for r in range(CHUNK)
        ]
        prow = t * CHUNK

        def jbody(j, acc):
            dsl = pl.ds(j * LANES, LANES)
            t0 = tt2_v[0, dsl]
            d1 = tt2_v[1, dsl]
            for r in range(CHUNK):
                tokq[r, dsl] = (tokq[r, dsl] + pos_v[prow + r, dsl]
                                + (t0 + fvecs[r] * d1))
            return acc

        lax.fori_loop(0, NJ, jbody, 0)

        flat_off = b * SEQ + sbase + t * CHUNK
        pltpu.make_async_copy(tokq, out_hbm.at[pl.ds(flat_off, CHUNK)],
                              osems[k]).start()

    for q in range(NBUF - 1):
        gather_start(q, q)

    def quad_body(i, acc):
        for k in range(NBUF):
            step(NBUF * i + k, k)
        return acc

    lax.fori_loop(0, NCHUNKS // NBUF, quad_body, 0)

    for q in range(NBUF):
        out_wait(q)


@jax.jit
def _emb_call(ids, ttids, token_table, token_type_table, pos_enc):
    mesh = plsc.VectorSubcoreMesh(core_axis_name="c", subcore_axis_name="s")
    f = pl.kernel(
        _emb_body,
        mesh=mesh,
        out_type=jax.ShapeDtypeStruct((ROWS, HIDDEN), jnp.float32),
        scratch_types=[
            pltpu.VMEM((ROWS_PER_W,), jnp.int32),
            pltpu.VMEM((ROWS_PER_W,), jnp.int32),
            pltpu.VMEM((S_PER_W, HIDDEN), jnp.float32),
            pltpu.VMEM((CHUNK, HIDDEN), jnp.float32),
            pltpu.VMEM((CHUNK, HIDDEN), jnp.float32),
            pltpu.VMEM((CHUNK, HIDDEN), jnp.float32),
            pltpu.VMEM((CHUNK, HIDDEN), jnp.float32),
            pltpu.VMEM((NUM_TYPES, HIDDEN), jnp.float32),
            pltpu.SemaphoreType.DMA,
            pltpu.SemaphoreType.DMA,
            pltpu.SemaphoreType.DMA,
            pltpu.SemaphoreType.DMA,
            pltpu.SemaphoreType.DMA,
            pltpu.SemaphoreType.DMA,
            pltpu.SemaphoreType.DMA,
            pltpu.SemaphoreType.DMA,
        ],
    )
    return f(ids, ttids, token_table, token_type_table, pos_enc)


def kernel(input_ids, token_type_ids, token_table, token_type_table, pos_enc):
    B, S = input_ids.shape
    ids = input_ids.reshape(-1).astype(jnp.int32)
    ttids = token_type_ids.reshape(-1).astype(jnp.int32)
    out = _emb_call(ids, ttids, token_table.astype(jnp.float32),
                    token_type_table.astype(jnp.float32),
                    pos_enc.astype(jnp.float32))
    return out.reshape(B, S, HIDDEN)


# speedup vs baseline: 1.3935x; 1.3935x over previous
import functools

import jax
import jax.numpy as jnp
from jax import lax
from jax.experimental import pallas as pl
from jax.experimental.pallas import tpu as pltpu
from jax.experimental.pallas import tpu_sc as plsc

BATCH = 4
SEQ = 2048
HIDDEN = 1024
NUM_TYPES = 2
LANES = 16
NJ = HIDDEN // LANES

ROWS = BATCH * SEQ
NW = 32
ROWS_PER_W = ROWS // NW
S_PER_W = SEQ // NW
CHUNK = 16
NCHUNKS = ROWS_PER_W // CHUNK
SCH_PER_B = S_PER_W // CHUNK
NBUF = 3


def _emb_body(ids_hbm, ttids_hbm, table_hbm, tt2_hbm, pos_hbm, out_hbm,
              idx_all, tid_all, pos_v, tok0, tok1, tok2, tt2_v,
              g0, g1, g2, o0, o1, o2, psem):
    nc = lax.axis_size("c")
    wid = lax.axis_index("s") * nc + lax.axis_index("c")
    sbase = wid * S_PER_W

    toks = [tok0, tok1, tok2]
    gsems = [g0, g1, g2]
    osems = [o0, o1, o2]

    for b in range(BATCH):
        off = b * SEQ + sbase
        pltpu.sync_copy(ids_hbm.at[pl.ds(off, S_PER_W)],
                        idx_all.at[pl.ds(b * S_PER_W, S_PER_W)])
        pltpu.sync_copy(ttids_hbm.at[pl.ds(off, S_PER_W)],
                        tid_all.at[pl.ds(b * S_PER_W, S_PER_W)])
    pltpu.sync_copy(tt2_hbm, tt2_v)
    pltpu.make_async_copy(pos_hbm.at[pl.ds(sbase, S_PER_W)], pos_v,
                          psem).start()

    def gather_start(cur):
        q = cur % NBUF
        pltpu.make_async_copy(
            table_hbm.at[idx_all.at[pl.ds(cur * CHUNK, CHUNK)]],
            toks[q], gsems[q]).start()

    def out_wait(q):
        pltpu.make_async_copy(toks[q], out_hbm.at[pl.ds(0, CHUNK)],
                              osems[q]).wait()

    gather_start(0)
    gather_start(1)
    for j in range(NJ):
        dsl = pl.ds(j * LANES, LANES)
        tt2_v[1, dsl] = tt2_v[1, dsl] - tt2_v[0, dsl]
    pltpu.make_async_copy(pos_hbm.at[pl.ds(0, S_PER_W)], pos_v, psem).wait()

    for cur in range(NCHUNKS):
        k = cur % NBUF
        if cur + 2 < NCHUNKS:
            if cur >= 1:
                out_wait((cur + 2) % NBUF)
            gather_start(cur + 2)

        pltpu.make_async_copy(
            table_hbm.at[idx_all.at[pl.ds(0, CHUNK)]],
            toks[k], gsems[k]).wait()

        b = cur // SCH_PER_B
        t = cur % SCH_PER_B
        tokq = toks[k]
        ttf = tid_all[pl.ds(cur * CHUNK, CHUNK)].astype(jnp.float32)
        fvecs = [
            ttf.at[jnp.full((LANES,), r, jnp.int32)].get(
                mode="promise_in_bounds") for r in range(CHUNK)
        ]
        prow = t * CHUNK

        def jbody(j, acc, tokq=tokq, fvecs=fvecs, prow=prow):
            dsl = pl.ds(j * LANES, LANES)
            t0 = tt2_v[0, dsl]
            d1 = tt2_v[1, dsl]
            for r in range(CHUNK):
                tokq[r, dsl] = (tokq[r, dsl] + pos_v[prow + r, dsl]
                                + (t0 + fvecs[r] * d1))
            return acc

        lax.fori_loop(0, NJ, jbody, 0)

        flat_off = b * SEQ + sbase + t * CHUNK
        pltpu.make_async_copy(tokq, out_hbm.at[pl.ds(flat_off, CHUNK)],
                              osems[k]).start()

    for q in range(NBUF):
        out_wait(q)


@jax.jit
def _emb_call(ids, ttids, token_table, token_type_table, pos_enc):
    mesh = plsc.VectorSubcoreMesh(core_axis_name="c", subcore_axis_name="s")
    f = pl.kernel(
        _emb_body,
        mesh=mesh,
        out_type=jax.ShapeDtypeStruct((ROWS, HIDDEN), jnp.float32),
        scratch_types=[
            pltpu.VMEM((ROWS_PER_W,), jnp.int32),
            pltpu.VMEM((ROWS_PER_W,), jnp.int32),
            pltpu.VMEM((S_PER_W, HIDDEN), jnp.float32),
            pltpu.VMEM((CHUNK, HIDDEN), jnp.float32),
            pltpu.VMEM((CHUNK, HIDDEN), jnp.float32),
            pltpu.VMEM((CHUNK, HIDDEN), jnp.float32),
            pltpu.VMEM((NUM_TYPES, HIDDEN), jnp.float32),
            pltpu.SemaphoreType.DMA,
            pltpu.SemaphoreType.DMA,
            pltpu.SemaphoreType.DMA,
            pltpu.SemaphoreType.DMA,
            pltpu.SemaphoreType.DMA,
            pltpu.SemaphoreType.DMA,
            pltpu.SemaphoreType.DMA,
        ],
    )
    return f(ids, ttids, token_table, token_type_table, pos_enc)


def kernel(input_ids, token_type_ids, token_table, token_type_table, pos_enc):
    B, S = input_ids.shape
    ids = input_ids.reshape(-1).astype(jnp.int32)
    ttids = token_type_ids.reshape(-1).astype(jnp.int32)
    out = _emb_call(ids, ttids, token_table.astype(jnp.float32),
                    token_type_table.astype(jnp.float32),
                    pos_enc.astype(jnp.float32))
    return out.reshape(B, S, HIDDEN)


# speedup vs baseline: 1.8696x; 1.3416x over previous
import functools

import jax
import jax.numpy as jnp
from jax import lax
from jax.experimental import pallas as pl
from jax.experimental.pallas import tpu as pltpu
from jax.experimental.pallas import tpu_sc as plsc

BATCH = 4
SEQ = 2048
HIDDEN = 1024
NUM_TYPES = 2
LANES = 16
NJ = HIDDEN // LANES

ROWS = BATCH * SEQ
NW = 32
ROWS_PER_W = ROWS // NW
S_PER_W = SEQ // NW
CHUNK = 16
NWIN = S_PER_W // CHUNK


def _emb_body(ids_hbm, ttids_hbm, table_hbm, tt2_hbm, pos_hbm, out_hbm,
              idx_all, tid_all, tok0, tok1, ob0, ob1, posA, posB, tt2_v,
              g0, g1, o0, o1, pA, pB):
    nc = lax.axis_size("c")
    wid = lax.axis_index("s") * nc + lax.axis_index("c")
    sbase = wid * S_PER_W

    toks = [tok0, tok1]
    obufs = [ob0, ob1]
    gsems = [g0, g1]
    osems = [o0, o1]

    for b in range(BATCH):
        off = b * SEQ + sbase
        pltpu.sync_copy(ids_hbm.at[pl.ds(off, S_PER_W)],
                        idx_all.at[pl.ds(b * S_PER_W, S_PER_W)])
        pltpu.sync_copy(ttids_hbm.at[pl.ds(off, S_PER_W)],
                        tid_all.at[pl.ds(b * S_PER_W, S_PER_W)])
    pltpu.sync_copy(tt2_hbm, tt2_v)

    def pos_start(t, posb, psem):
        pltpu.make_async_copy(pos_hbm.at[pl.ds(sbase + t * CHUNK, CHUNK)],
                              posb, psem).start()

    def pos_wait(posb, psem):
        pltpu.make_async_copy(pos_hbm.at[pl.ds(0, CHUNK)], posb, psem).wait()

    def gather_start(t, b, p):
        idxvec = idx_all[pl.ds(b * S_PER_W + t * CHUNK, CHUNK)]
        pltpu.make_async_copy(table_hbm.at[idxvec], toks[p], gsems[p]).start()

    pos_start(0, posA, pA)
    gather_start(0, 0, 0)
    for j in range(NJ):
        dsl = pl.ds(j * LANES, LANES)
        tt2_v[1, dsl] = tt2_v[1, dsl] - tt2_v[0, dsl]

    def step(t, b, posbuf):
        p = b % 2
        if b < BATCH - 1:
            gather_start(t, b + 1, 1 - p)
        else:
            @pl.when(t + 1 < NWIN)
            def _():
                gather_start(t + 1, 0, 1 - p)

        pltpu.make_async_copy(table_hbm.at[idx_all[pl.ds(0, CHUNK)]],
                              toks[p], gsems[p]).wait()

        @pl.when(t * BATCH + b >= 2)
        def _():
            pltpu.make_async_copy(obufs[p], out_hbm.at[pl.ds(0, CHUNK)],
                                  osems[p]).wait()

        tokq = toks[p]
        obq = obufs[p]
        ttf = tid_all[pl.ds(b * S_PER_W + t * CHUNK, CHUNK)].astype(
            jnp.float32)
        fvecs = [
            ttf.at[jnp.full((LANES,), r, jnp.int32)].get(
                mode="promise_in_bounds") for r in range(CHUNK)
        ]

        def jbody(j, acc):
            dsl = pl.ds(j * LANES, LANES)
            t0 = tt2_v[0, dsl]
            d1 = tt2_v[1, dsl]
            for r in range(CHUNK):
                obq[r, dsl] = (tokq[r, dsl] + posbuf[r, dsl]
                               + (t0 + fvecs[r] * d1))
            return acc

        lax.fori_loop(0, NJ, jbody, 0)

        flat_off = b * SEQ + sbase + t * CHUNK
        pltpu.make_async_copy(obq, out_hbm.at[pl.ds(flat_off, CHUNK)],
                              osems[p]).start()

    def body(i, acc):
        t0 = 2 * i
        t1 = t0 + 1
        pos_start(t1, posB, pB)
        pos_wait(posA, pA)
        for b in range(BATCH):
            step(t0, b, posA)
        @pl.when(t1 + 1 < NWIN)
        def _():
            pos_start(t1 + 1, posA, pA)

        pos_wait(posB, pB)
        for b in range(BATCH):
            step(t1, b, posB)
        return acc

    lax.fori_loop(0, NWIN // 2, body, 0)

    pltpu.make_async_copy(ob0, out_hbm.at[pl.ds(0, CHUNK)], o0).wait()
    pltpu.make_async_copy(ob1, out_hbm.at[pl.ds(0, CHUNK)], o1).wait()


@jax.jit
def _emb_call(ids, ttids, token_table, token_type_table, pos_enc):
    mesh = plsc.VectorSubcoreMesh(core_axis_name="c", subcore_axis_name="s")
    f = pl.kernel(
        _emb_body,
        mesh=mesh,
        out_type=jax.ShapeDtypeStruct((ROWS, HIDDEN), jnp.float32),
        scratch_types=[
            pltpu.VMEM((ROWS_PER_W,), jnp.int32),
            pltpu.VMEM((ROWS_PER_W,), jnp.int32),
            pltpu.VMEM((CHUNK, HIDDEN), jnp.float32),
            pltpu.VMEM((CHUNK, HIDDEN), jnp.float32),
            pltpu.VMEM((CHUNK, HIDDEN), jnp.float32),
            pltpu.VMEM((CHUNK, HIDDEN), jnp.float32),
            pltpu.VMEM((CHUNK, HIDDEN), jnp.float32),
            pltpu.VMEM((CHUNK, HIDDEN), jnp.float32),
            pltpu.VMEM((NUM_TYPES, HIDDEN), jnp.float32),
            pltpu.SemaphoreType.DMA,
            pltpu.SemaphoreType.DMA,
            pltpu.SemaphoreType.DMA,
            pltpu.SemaphoreType.DMA,
            pltpu.SemaphoreType.DMA,
            pltpu.SemaphoreType.DMA,
        ],
    )
    return f(ids, ttids, token_table, token_type_table, pos_enc)


def kernel(input_ids, token_type_ids, token_table, token_type_table, pos_enc):
    B, S = input_ids.shape
    ids = input_ids.reshape(-1).astype(jnp.int32)
    ttids = token_type_ids.reshape(-1).astype(jnp.int32)
    out = _emb_call(ids, ttids, token_table.astype(jnp.float32),
                    token_type_table.astype(jnp.float32),
                    pos_enc.astype(jnp.float32))
    return out.reshape(B, S, HIDDEN)
